# Initial kernel scaffold; baseline (speedup 1.0000x reference)
import jax
import jax.numpy as jnp
from jax.experimental import pallas as pl


def kernel(feat_index, feat_value, first_table, emb_table, W0, b0, g0, be0, W1, b1, g1, be1, W2, b2, g2, be2, Wfc, bfc):
    raise NotImplementedError("write your pallas kernel here")


# speedup vs baseline: 2.4798x; 2.4798x over previous
import functools

import jax
import jax.numpy as jnp
from jax import lax
from jax.experimental import pallas as pl
from jax.experimental.pallas import tpu as pltpu
from jax.experimental.pallas import tpu_sc as plsc

B, F, V, D = 4096, 26, 100000, 32
L0 = F * D
H = 400
EPS = 1e-3

NC, NS = 2, 16
NW = NC * NS
BF = B * F
ROWS_PER_W = BF // NW
CHUNK = 128
CHUNKS = ROWS_PER_W // CHUNK


def _sc_gather(idx3, emb_table, first_table):
  mesh = plsc.VectorSubcoreMesh(core_axis_name="c", subcore_axis_name="s")

  @functools.partial(
      pl.kernel,
      mesh=mesh,
      out_type=(
          jax.ShapeDtypeStruct((BF, D), jnp.float32),
          jax.ShapeDtypeStruct((BF,), jnp.float32),
      ),
      scratch_types=[
          pltpu.VMEM((CHUNKS, CHUNK), jnp.int32),
          pltpu.VMEM((ROWS_PER_W, D), jnp.float32),
          pltpu.VMEM((ROWS_PER_W,), jnp.float32),
          pltpu.SemaphoreType.DMA,
          pltpu.SemaphoreType.DMA,
      ],
      compiler_params=pltpu.CompilerParams(use_tc_tiling_on_sc=False),
  )
  def k(idx_hbm, emb_hbm, first_hbm, out_rows, out_fw, idx_v, rows_v, fw_v,
        sem_e, sem_f):
    wid = lax.axis_index("s") * NC + lax.axis_index("c")
    base = wid * ROWS_PER_W
    pltpu.sync_copy(idx_hbm.at[wid], idx_v)
    copies = []
    for j in range(CHUNKS):
      copies.append(pltpu.async_copy(
          emb_hbm.at[idx_v.at[j]], rows_v.at[pl.ds(j * CHUNK, CHUNK)], sem_e))
      copies.append(pltpu.async_copy(
          first_hbm.at[idx_v.at[j]], fw_v.at[pl.ds(j * CHUNK, CHUNK)], sem_f))
    for c in copies:
      c.wait()
    pltpu.sync_copy(rows_v, out_rows.at[pl.ds(base, ROWS_PER_W)])
    pltpu.sync_copy(fw_v, out_fw.at[pl.ds(base, ROWS_PER_W)])

  return k(idx3, emb_table, first_table)


def _dense_body(emb_ref, fv_ref, fw_ref,
                w0_ref, b0_ref, w1_ref, b1_ref, w2_ref, b2_ref,
                wfc1_ref, wfc2_ref, wfc3_ref, bfc_ref, out_ref):
  f32 = jnp.float32
  colsE = lax.broadcasted_iota(jnp.int32, (F, L0), 1)
  rowsE = lax.broadcasted_iota(jnp.int32, (F, L0), 0)
  E = (colsE // D == rowsE).astype(f32)
  rowsS = lax.broadcasted_iota(jnp.int32, (L0, D), 0)
  colsS = lax.broadcasted_iota(jnp.int32, (L0, D), 1)
  S = (rowsS % D == colsS).astype(f32)

  fv = fv_ref[...]
  emb_w = emb_ref[...] * jnp.dot(fv, E, preferred_element_type=f32)

  summed = jnp.dot(emb_w, S, preferred_element_type=f32)
  part2 = jnp.dot(emb_w * emb_w, S, preferred_element_type=f32)
  y2 = 0.5 * (summed * summed - part2)
  y1 = fw_ref[...] * fv
  h = emb_w
  for w_ref, b_ref in ((w0_ref, b0_ref), (w1_ref, b1_ref), (w2_ref, b2_ref)):
    h = jnp.dot(h, w_ref[...], preferred_element_type=f32) + b_ref[...]
    h = jnp.maximum(h, 0.0)
  logit = (jnp.dot(y1, wfc1_ref[...], preferred_element_type=f32)
           + jnp.dot(y2, wfc2_ref[...], preferred_element_type=f32)
           + jnp.dot(h, wfc3_ref[...], preferred_element_type=f32)
           + bfc_ref[0, 0])
  out_ref[...] = 1.0 / (1.0 + jnp.exp(-logit))


def _dense(emb_g, fv, fw, w0, b0, w1, b1, w2, b2, wfc1, wfc2, wfc3, bfc):
  BB = 1024
  grid = (B // BB,)
  bs = lambda shp: pl.BlockSpec(shp, lambda i: (0,) * len(shp))
  return pl.pallas_call(
      _dense_body,
      grid=grid,
      in_specs=[
          pl.BlockSpec((BB, L0), lambda i: (i, 0)),
          pl.BlockSpec((BB, F), lambda i: (i, 0)),
          pl.BlockSpec((BB, F), lambda i: (i, 0)),
          bs((L0, H)), bs((1, H)),
          bs((H, H)), bs((1, H)),
          bs((H, H)), bs((1, H)),
          bs((F, 1)), bs((D, 1)), bs((H, 1)), bs((1, 1)),
      ],
      out_specs=pl.BlockSpec((BB, 1), lambda i: (i, 0)),
      out_shape=jax.ShapeDtypeStruct((B, 1), jnp.float32),
  )(emb_g, fv, fw, w0, b0, w1, b1, w2, b2, wfc1, wfc2, wfc3, bfc)


def kernel(feat_index, feat_value, first_table, emb_table,
           W0, b0, g0, be0, W1, b1, g1, be1, W2, b2, g2, be2, Wfc, bfc):
  idx3 = feat_index.reshape(NW, CHUNKS, CHUNK).astype(jnp.int32)
  rows, fw = _sc_gather(idx3, emb_table, first_table.reshape(V))
  emb_g = rows.reshape(B, L0)
  fw2 = fw.reshape(B, F)

  inv = (1.0 / jnp.sqrt(jnp.float32(1.0 + EPS)))
  s0, s1, s2 = g0 * inv, g1 * inv, g2 * inv
  w0f, b0f = W0 * s0[None, :], (b0 * s0 + be0)[None, :]
  w1f, b1f = W1 * s1[None, :], (b1 * s1 + be1)[None, :]
  w2f, b2f = W2 * s2[None, :], (b2 * s2 + be2)[None, :]

  wfc1 = Wfc[:F]
  wfc2 = Wfc[F:F + D]
  wfc3 = Wfc[F + D:]
  return _dense(emb_g, feat_value, fw2, w0f, b0f, w1f, b1f, w2f, b2f,
                wfc1, wfc2, wfc3, bfc.reshape(1, 1))
